# baseline (speedup 1.0000x reference)
import functools

import jax
import jax.numpy as jnp
from jax import lax
from jax.experimental import pallas as pl
from jax.experimental.pallas import tpu as pltpu
import jax.experimental.pallas.tpu_sc as plsc

N_N = 10000
N_E = 320000
F_IN = 128
F_H = 64
F_OUT = 2

NC = 2
NS = 16
CH = 80
N_P = 10240
E_ROWS = N_E // CH
NCH = E_ROWS // (NC * NS)
RPW = N_P // NS
SCH = RPW // CH
DW = 16
NG = NCH
NV = F_H // 16

_SC_MESH = plsc.VectorSubcoreMesh(core_axis_name="c", subcore_axis_name="s")
_SC_PARAMS = pltpu.CompilerParams(use_tc_tiling_on_sc=False,
                                  needs_layout_passes=False)


def _rsqrt16(d):
    i = plsc.bitcast(d, jnp.int32)
    i = jnp.int32(0x5F3759DF) - (i >> 1)
    y = plsc.bitcast(i, jnp.float32)
    y = y * (1.5 - 0.5 * d * y * y)
    y = y * (1.5 - 0.5 * d * y * y)
    return y


@functools.partial(
    pl.kernel,
    out_type=jax.ShapeDtypeStruct((NC, N_P, DW), jnp.float32),
    mesh=_SC_MESH,
    scratch_types=[
        pltpu.VMEM((NCH, CH), jnp.int32),
        pltpu.VMEM((CH, DW), jnp.float32),
        pltpu.VMEM_SHARED((N_P, DW), jnp.float32),
    ],
    compiler_params=_SC_PARAMS,
)
def _deg_sc(edge_hbm, out_hbm, dst_v, ones_v, deg_sh):
    cid = lax.axis_index("c")
    sid = lax.axis_index("s")
    row0 = sid * RPW
    cbase = (cid * NS + sid) * NCH

    def fill0(r, carry):
        ones_v[r, :] = jnp.zeros((DW,), jnp.float32)
        return carry

    lax.fori_loop(0, CH, fill0, 0)
    for t in range(SCH):
        pltpu.sync_copy(ones_v, deg_sh.at[pl.ds(row0 + t * CH, CH)])

    def fill1(r, carry):
        ones_v[r, :] = jnp.ones((DW,), jnp.float32)
        return carry

    lax.fori_loop(0, CH, fill1, 0)
    pltpu.sync_copy(edge_hbm.at[1, pl.ds(cbase, NCH)], dst_v)
    plsc.subcore_barrier()

    def body(i, carry):
        pltpu.sync_copy(ones_v, deg_sh.at[dst_v.at[i]], add=True)
        return carry

    lax.fori_loop(0, NCH, body, 0)
    plsc.subcore_barrier()
    pltpu.sync_copy(deg_sh.at[pl.ds(row0, RPW)], out_hbm.at[cid, pl.ds(row0, RPW)])


def _edge_pipeline(edge_hbm, cbase, g_sh, agg_sh,
                   src_v, dst_v, rows0, rows1, sg0, sg1, ss0, ss1):
    pltpu.sync_copy(edge_hbm.at[0, pl.ds(cbase, NCH)], src_v)
    pltpu.sync_copy(edge_hbm.at[1, pl.ds(cbase, NCH)], dst_v)

    def gather_issue(g, rows, sg):
        pltpu.async_copy(g_sh.at[src_v.at[g]], rows, sg)

    def gather_wait(g, rows, sg):
        pltpu.make_async_copy(g_sh.at[src_v.at[g]], rows, sg).wait()

    def scatter_issue(g, rows, ss):
        pltpu.async_copy(rows, agg_sh.at[dst_v.at[g]], ss, add=True)

    def scatter_wait(g, rows, ss):
        pltpu.make_async_copy(rows, agg_sh.at[dst_v.at[g]], ss).wait()

    halves = ((rows0, sg0, ss0), (rows1, sg1, ss1))
    gather_issue(0, rows0, sg0)

    def pair(p, carry):
        for half in range(2):
            rows, sg, ss = halves[half]
            orows, osg, oss = halves[1 - half]
            g = 2 * p + half
            gather_wait(g, rows, sg)
            scatter_issue(g, rows, ss)

            @pl.when(g >= 1)
            def _():
                scatter_wait(g - 1, orows, oss)

            @pl.when(g + 1 < NG)
            def _():
                gather_issue(g + 1, orows, osg)
        return carry

    lax.fori_loop(0, NG // 2, pair, 0)
    gather_wait(NG - 1, rows0, sg0)
    scatter_issue(NG - 1, rows0, ss0)
    scatter_wait(NG - 2, rows1, ss1)
    scatter_wait(NG - 1, rows0, ss0)


def _zero_agg(row0, ubuf, agg_sh):
    def fillz(r4, carry):
        for u in range(4):
            for j in range(NV):
                ubuf[r4 * 4 + u, pl.ds(j * 16, 16)] = jnp.zeros((16,), jnp.float32)
        return carry

    lax.fori_loop(0, CH // 4, fillz, 0)
    for t in range(SCH):
        pltpu.sync_copy(ubuf, agg_sh.at[pl.ds(row0 + t * CH, CH)])


@functools.partial(
    pl.kernel,
    out_type=[jax.ShapeDtypeStruct((NC, N_P, F_H), jnp.float32),
              jax.ShapeDtypeStruct((N_P, DW), jnp.float32)],
    mesh=_SC_MESH,
    scratch_types=[
        pltpu.VMEM((NCH, CH), jnp.int32),
        pltpu.VMEM((NCH, CH), jnp.int32),
        pltpu.VMEM((CH, F_H), jnp.float32),
        pltpu.VMEM((CH, F_H), jnp.float32),
        pltpu.VMEM((CH, F_H), jnp.float32),
        pltpu.VMEM((CH, DW), jnp.float32),
        pltpu.VMEM((CH, DW), jnp.float32),
        pltpu.VMEM_SHARED((N_P, F_H), jnp.float32),
        pltpu.VMEM_SHARED((N_P, F_H), jnp.float32),
        pltpu.SemaphoreType.DMA,
        pltpu.SemaphoreType.DMA,
        pltpu.SemaphoreType.DMA,
        pltpu.SemaphoreType.DMA,
    ],
    compiler_params=_SC_PARAMS,
)
def _prop1_sc(h_hbm, deg2_hbm, edge_hbm, agg_out, dis_out,
              src_v, dst_v, rows0, rows1, ubuf, dbuf, dbuf2,
              g_sh, agg_sh, sg0, sg1, ss0, ss1):
    cid = lax.axis_index("c")
    sid = lax.axis_index("s")
    row0 = sid * RPW
    cbase = (cid * NS + sid) * NCH

    _zero_agg(row0, ubuf, agg_sh)
    for t in range(SCH):
        r0 = row0 + t * CH
        pltpu.sync_copy(h_hbm.at[pl.ds(r0, CH)], rows0)
        pltpu.sync_copy(deg2_hbm.at[0, pl.ds(r0, CH)], dbuf)
        pltpu.sync_copy(deg2_hbm.at[1, pl.ds(r0, CH)], dbuf2)

        def srow(r4, carry):
            for u in range(4):
                r = r4 * 4 + u
                d = jnp.maximum(dbuf[r, :] + dbuf2[r, :], 1.0)
                y = _rsqrt16(d)
                dbuf[r, :] = y
                for j in range(NV):
                    rows1[r, pl.ds(j * 16, 16)] = rows0[r, pl.ds(j * 16, 16)] * y
            return carry

        lax.fori_loop(0, CH // 4, srow, 0)
        pltpu.sync_copy(rows1, g_sh.at[pl.ds(r0, CH)])

        @pl.when(cid == 0)
        def _():
            pltpu.sync_copy(dbuf, dis_out.at[pl.ds(r0, CH)])

    plsc.subcore_barrier()
    _edge_pipeline(edge_hbm, cbase, g_sh, agg_sh,
                   src_v, dst_v, rows0, rows1, sg0, sg1, ss0, ss1)
    plsc.subcore_barrier()
    pltpu.sync_copy(agg_sh.at[pl.ds(row0, RPW)], agg_out.at[cid, pl.ds(row0, RPW)])


@functools.partial(
    pl.kernel,
    out_type=[jax.ShapeDtypeStruct((NC, N_P, F_H), jnp.float32),
              jax.ShapeDtypeStruct((N_P, F_H), jnp.float32)],
    mesh=_SC_MESH,
    scratch_types=[
        pltpu.VMEM((NCH, CH), jnp.int32),
        pltpu.VMEM((NCH, CH), jnp.int32),
        pltpu.VMEM((CH, F_H), jnp.float32),
        pltpu.VMEM((CH, F_H), jnp.float32),
        pltpu.VMEM((CH, F_H), jnp.float32),
        pltpu.VMEM((CH, DW), jnp.float32),
        pltpu.VMEM_SHARED((N_P, F_H), jnp.float32),
        pltpu.VMEM_SHARED((N_P, F_H), jnp.float32),
        pltpu.SemaphoreType.DMA,
        pltpu.SemaphoreType.DMA,
        pltpu.SemaphoreType.DMA,
        pltpu.SemaphoreType.DMA,
    ],
    compiler_params=_SC_PARAMS,
)
def _prop2_sc(h_hbm, dis_hbm, agg1_hbm, edge_hbm, agg_out, p1_out,
              src_v, dst_v, rows0, rows1, ubuf, dbuf,
              g_sh, agg_sh, sg0, sg1, ss0, ss1):
    cid = lax.axis_index("c")
    sid = lax.axis_index("s")
    row0 = sid * RPW
    cbase = (cid * NS + sid) * NCH

    for t in range(SCH):
        r0 = row0 + t * CH
        pltpu.sync_copy(h_hbm.at[pl.ds(r0, CH)], rows0)
        pltpu.sync_copy(agg1_hbm.at[0, pl.ds(r0, CH)], rows1)
        pltpu.sync_copy(agg1_hbm.at[1, pl.ds(r0, CH)], ubuf)
        pltpu.sync_copy(dis_hbm.at[pl.ds(r0, CH)], dbuf)

        def srow(r4, carry):
            for u in range(4):
                r = r4 * 4 + u
                y = dbuf[r, :]
                for j in range(NV):
                    a = rows1[r, pl.ds(j * 16, 16)] + ubuf[r, pl.ds(j * 16, 16)]
                    p = rows0[r, pl.ds(j * 16, 16)] - y * a
                    rows1[r, pl.ds(j * 16, 16)] = p
                    ubuf[r, pl.ds(j * 16, 16)] = y * p
            return carry

        lax.fori_loop(0, CH // 4, srow, 0)
        pltpu.sync_copy(ubuf, g_sh.at[pl.ds(r0, CH)])

        @pl.when(cid == 0)
        def _():
            pltpu.sync_copy(rows1, p1_out.at[pl.ds(r0, CH)])

    _zero_agg(row0, ubuf, agg_sh)
    plsc.subcore_barrier()
    _edge_pipeline(edge_hbm, cbase, g_sh, agg_sh,
                   src_v, dst_v, rows0, rows1, sg0, sg1, ss0, ss1)
    plsc.subcore_barrier()
    pltpu.sync_copy(agg_sh.at[pl.ds(row0, RPW)], agg_out.at[cid, pl.ds(row0, RPW)])


BR = 1000


def _mlp_body(x_ref, w1_ref, b1_ref, w2_ref, b2_ref, o_ref):
    h1 = jnp.dot(x_ref[...], w1_ref[...], preferred_element_type=jnp.float32)
    h1 = jnp.maximum(h1 + b1_ref[...], 0.0)
    h2 = jnp.dot(h1, w2_ref[...], preferred_element_type=jnp.float32)
    o_ref[...] = jnp.maximum(h2 + b2_ref[...], 0.0)


def _final_body(h_ref, p1_ref, dis_ref, agg_ref, w3_ref, b3_ref, w4_ref,
                b4_ref, y_ref):
    dis = dis_ref[:, 0:1]
    h = h_ref[...]
    p1 = p1_ref[...]
    p2 = p1 - dis * (agg_ref[0] + agg_ref[1])
    c0 = 3.0 * h - 3.0 * p1 + 0.75 * p2
    c1 = 3.0 * p1 - 1.5 * p2
    c2 = 0.75 * p2
    acts = jnp.dot(c0, w3_ref[0:F_H, :], preferred_element_type=jnp.float32)
    acts += jnp.dot(c1, w3_ref[F_H:2 * F_H, :], preferred_element_type=jnp.float32)
    acts += jnp.dot(c2, w3_ref[2 * F_H:3 * F_H, :], preferred_element_type=jnp.float32)
    acts = jnp.maximum(acts + b3_ref[...], 0.0)
    y = jnp.dot(acts, w4_ref[...], preferred_element_type=jnp.float32)
    y_ref[...] = y + b4_ref[...]


def _row_spec(width):
    return pl.BlockSpec((BR, width), lambda i: (i, 0))


def _full_spec(shape):
    nd = len(shape)
    return pl.BlockSpec(shape, lambda i, _nd=nd: (0,) * _nd)


def _agg_spec(width):
    return pl.BlockSpec((NC, BR, width), lambda i: (0, i, 0))


def kernel(x, edge_index, W1, b1, W2, b2, W3, b3, W4, b4):
    edge3d = edge_index.astype(jnp.int32).reshape(2, E_ROWS, CH)
    b1r = b1.reshape(1, F_H)
    b2r = b2.reshape(1, F_H)
    b3r = b3.reshape(1, F_H)
    b4r = b4.reshape(1, F_OUT)

    grid = (N_N // BR,)

    deg2 = _deg_sc(edge3d)

    h = pl.pallas_call(
        _mlp_body,
        grid=grid,
        in_specs=[_row_spec(F_IN), _full_spec((F_IN, F_H)), _full_spec((1, F_H)),
                  _full_spec((F_H, F_H)), _full_spec((1, F_H))],
        out_specs=_row_spec(F_H),
        out_shape=jax.ShapeDtypeStruct((N_P, F_H), jnp.float32),
    )(x, W1, b1r, W2, b2r)

    agg1, dis = _prop1_sc(h, deg2, edge3d)
    agg2, p1 = _prop2_sc(h, dis, agg1, edge3d)

    y = pl.pallas_call(
        _final_body,
        grid=grid,
        in_specs=[_row_spec(F_H), _row_spec(F_H), _row_spec(DW),
                  _agg_spec(F_H), _full_spec((3 * F_H, F_H)),
                  _full_spec((1, F_H)), _full_spec((F_H, F_OUT)),
                  _full_spec((1, F_OUT))],
        out_specs=_row_spec(F_OUT),
        out_shape=jax.ShapeDtypeStruct((N_N, F_OUT), jnp.float32),
    )(h, p1, dis, agg2, W3, b3r, W4, b4r)

    return y


# speedup vs baseline: 1.1016x; 1.1016x over previous
import functools

import jax
import jax.numpy as jnp
from jax import lax
from jax.experimental import pallas as pl
from jax.experimental.pallas import tpu as pltpu
import jax.experimental.pallas.tpu_sc as plsc

N_N = 10000
N_E = 320000
F_IN = 128
F_H = 64
F_OUT = 2

NC = 2
NS = 16
CH = 80
N_P = 10240
E_ROWS = N_E // CH
NCH = E_ROWS // (NC * NS)
RPW = N_P // NS
DW = 8
GC = 1
NG = NCH // GC

_SC_MESH = plsc.VectorSubcoreMesh(core_axis_name="c", subcore_axis_name="s")
_SC_PARAMS = pltpu.CompilerParams(use_tc_tiling_on_sc=False)


@functools.partial(
    pl.kernel,
    out_type=jax.ShapeDtypeStruct((NC, N_P, DW), jnp.float32),
    mesh=_SC_MESH,
    scratch_types=[
        pltpu.VMEM((NCH, CH), jnp.int32),
        pltpu.VMEM((CH, DW), jnp.float32),
        pltpu.VMEM_SHARED((N_P, DW), jnp.float32),
    ],
    compiler_params=_SC_PARAMS,
)
def _deg_sc(edge_hbm, zeros_hbm, ones_hbm, out_hbm, dst_v, ones_v, deg_sh):
    cid = lax.axis_index("c")
    sid = lax.axis_index("s")
    row0 = sid * RPW
    cbase = (cid * NS + sid) * NCH
    pltpu.sync_copy(zeros_hbm.at[pl.ds(row0, RPW)], deg_sh.at[pl.ds(row0, RPW)])
    pltpu.sync_copy(ones_hbm, ones_v)
    pltpu.sync_copy(edge_hbm.at[1, pl.ds(cbase, NCH)], dst_v)
    plsc.subcore_barrier()

    def body(i, carry):
        pltpu.sync_copy(ones_v, deg_sh.at[dst_v.at[i]], add=True)
        return carry

    lax.fori_loop(0, NCH, body, 0)
    plsc.subcore_barrier()
    pltpu.sync_copy(deg_sh.at[pl.ds(row0, RPW)], out_hbm.at[cid, pl.ds(row0, RPW)])


@functools.partial(
    pl.kernel,
    out_type=jax.ShapeDtypeStruct((NC, N_P, F_H), jnp.float32),
    mesh=_SC_MESH,
    scratch_types=[
        pltpu.VMEM((NCH, CH), jnp.int32),
        pltpu.VMEM((NCH, CH), jnp.int32),
        pltpu.VMEM((GC, CH, F_H), jnp.float32),
        pltpu.VMEM((GC, CH, F_H), jnp.float32),
        pltpu.VMEM_SHARED((N_P, F_H), jnp.float32),
        pltpu.VMEM_SHARED((N_P, F_H), jnp.float32),
        pltpu.SemaphoreType.DMA,
        pltpu.SemaphoreType.DMA,
        pltpu.SemaphoreType.DMA,
        pltpu.SemaphoreType.DMA,
    ],
    compiler_params=_SC_PARAMS,
)
def _prop_sc(g_hbm_in, edge_hbm, zeros_hbm, out_hbm,
             src_v, dst_v, rows0, rows1, g_hbm, agg_sh, sg0, sg1, ss0, ss1):
    cid = lax.axis_index("c")
    sid = lax.axis_index("s")
    row0 = sid * RPW
    cbase = (cid * NS + sid) * NCH
    pltpu.sync_copy(zeros_hbm.at[pl.ds(row0, RPW)], agg_sh.at[pl.ds(row0, RPW)])
    pltpu.sync_copy(g_hbm_in.at[pl.ds(row0, RPW)], g_hbm.at[pl.ds(row0, RPW)])
    pltpu.sync_copy(edge_hbm.at[0, pl.ds(cbase, NCH)], src_v)
    pltpu.sync_copy(edge_hbm.at[1, pl.ds(cbase, NCH)], dst_v)
    plsc.subcore_barrier()

    def gather_issue(g, rows, sg):
        def c_body(c, carry):
            pltpu.async_copy(g_hbm.at[src_v.at[g * GC + c]], rows.at[c], sg)
            return carry
        lax.fori_loop(0, GC, c_body, 0)

    def gather_wait(g, rows, sg):
        def c_body(c, carry):
            pltpu.make_async_copy(
                g_hbm.at[src_v.at[g * GC + c]], rows.at[c], sg).wait()
            return carry
        lax.fori_loop(0, GC, c_body, 0)

    def scatter_issue(g, rows, ss):
        def c_body(c, carry):
            pltpu.async_copy(
                rows.at[c], agg_sh.at[dst_v.at[g * GC + c]], ss, add=True)
            return carry
        lax.fori_loop(0, GC, c_body, 0)

    def scatter_wait(g, rows, ss):
        def c_body(c, carry):
            pltpu.make_async_copy(
                rows.at[c], agg_sh.at[dst_v.at[g * GC + c]], ss).wait()
            return carry
        lax.fori_loop(0, GC, c_body, 0)

    halves = ((rows0, sg0, ss0), (rows1, sg1, ss1))
    gather_issue(0, rows0, sg0)

    def pair(p, carry):
        for half in range(2):
            rows, sg, ss = halves[half]
            orows, osg, oss = halves[1 - half]
            g = 2 * p + half
            gather_wait(g, rows, sg)
            scatter_issue(g, rows, ss)

            @pl.when(g >= 1)
            def _():
                scatter_wait(g - 1, orows, oss)

            @pl.when(g + 1 < NG)
            def _():
                gather_issue(g + 1, orows, osg)
        return carry

    lax.fori_loop(0, NG // 2, pair, 0)
    gather_wait(NG - 1, rows0, sg0)
    scatter_issue(NG - 1, rows0, ss0)
    scatter_wait(NG - 2, rows1, ss1)
    scatter_wait(NG - 1, rows0, ss0)
    plsc.subcore_barrier()
    pltpu.sync_copy(agg_sh.at[pl.ds(row0, RPW)], out_hbm.at[cid, pl.ds(row0, RPW)])


BR = 1000


def _mlp_body(x_ref, w1_ref, b1_ref, w2_ref, b2_ref, o_ref):
    h1 = jnp.dot(x_ref[...], w1_ref[...], preferred_element_type=jnp.float32)
    h1 = jnp.maximum(h1 + b1_ref[...], 0.0)
    h2 = jnp.dot(h1, w2_ref[...], preferred_element_type=jnp.float32)
    o_ref[...] = jnp.maximum(h2 + b2_ref[...], 0.0)


def _scale_body(deg_ref, h_ref, g_ref, dis_ref):
    deg = jnp.maximum(deg_ref[0] + deg_ref[1], 1.0)
    dis = lax.rsqrt(deg)
    dis_ref[...] = dis
    g_ref[...] = h_ref[...] * dis[:, 0:1]


def _update_body(h_ref, dis_ref, agg_ref, g_ref):
    dis = dis_ref[:, 0:1]
    p = h_ref[...] - dis * (agg_ref[0] + agg_ref[1])
    g_ref[...] = p * dis


def _final_body(h_ref, dis_ref, agg1_ref, agg2_ref, w3_ref, b3_ref, w4_ref,
                b4_ref, y_ref):
    dis = dis_ref[:, 0:1]
    h = h_ref[...]
    q1 = dis * (agg1_ref[0] + agg1_ref[1])
    q2 = dis * (agg2_ref[0] + agg2_ref[1])
    c0 = 0.75 * h + 2.25 * q1 - 0.75 * q2
    c1 = 1.5 * h - 1.5 * q1 + 1.5 * q2
    c2 = 0.75 * h - 0.75 * q1 - 0.75 * q2
    acts = jnp.dot(c0, w3_ref[0:F_H, :], preferred_element_type=jnp.float32)
    acts += jnp.dot(c1, w3_ref[F_H:2 * F_H, :], preferred_element_type=jnp.float32)
    acts += jnp.dot(c2, w3_ref[2 * F_H:3 * F_H, :], preferred_element_type=jnp.float32)
    acts = jnp.maximum(acts + b3_ref[...], 0.0)
    y = jnp.dot(acts, w4_ref[...], preferred_element_type=jnp.float32)
    y_ref[...] = y + b4_ref[...]


def _row_spec(width):
    return pl.BlockSpec((BR, width), lambda i: (i, 0))


def _full_spec(shape):
    nd = len(shape)
    return pl.BlockSpec(shape, lambda i, _nd=nd: (0,) * _nd)


def _agg_spec(width):
    return pl.BlockSpec((NC, BR, width), lambda i: (0, i, 0))


def kernel(x, edge_index, W1, b1, W2, b2, W3, b3, W4, b4):
    edge3d = edge_index.astype(jnp.int32).reshape(2, E_ROWS, CH)
    zeros_h = jnp.zeros((N_P, F_H), jnp.float32)
    zeros_d = jnp.zeros((N_P, DW), jnp.float32)
    ones_d = jnp.ones((CH, DW), jnp.float32)
    b1r = b1.reshape(1, F_H)
    b2r = b2.reshape(1, F_H)
    b3r = b3.reshape(1, F_H)
    b4r = b4.reshape(1, F_OUT)

    grid = (N_N // BR,)

    deg2 = _deg_sc(edge3d, zeros_d, ones_d)

    h = pl.pallas_call(
        _mlp_body,
        grid=grid,
        in_specs=[_row_spec(F_IN), _full_spec((F_IN, F_H)), _full_spec((1, F_H)),
                  _full_spec((F_H, F_H)), _full_spec((1, F_H))],
        out_specs=_row_spec(F_H),
        out_shape=jax.ShapeDtypeStruct((N_N, F_H), jnp.float32),
    )(x, W1, b1r, W2, b2r)

    g1, dis = pl.pallas_call(
        _scale_body,
        grid=grid,
        in_specs=[_agg_spec(DW), _row_spec(F_H)],
        out_specs=[_row_spec(F_H), _row_spec(DW)],
        out_shape=[jax.ShapeDtypeStruct((N_P, F_H), jnp.float32),
                   jax.ShapeDtypeStruct((N_N, DW), jnp.float32)],
    )(deg2, h)

    agg1 = _prop_sc(g1, edge3d, zeros_h)

    g2 = pl.pallas_call(
        _update_body,
        grid=grid,
        in_specs=[_row_spec(F_H), _row_spec(DW), _agg_spec(F_H)],
        out_specs=_row_spec(F_H),
        out_shape=jax.ShapeDtypeStruct((N_P, F_H), jnp.float32),
    )(h, dis, agg1)

    agg2 = _prop_sc(g2, edge3d, zeros_h)

    y = pl.pallas_call(
        _final_body,
        grid=grid,
        in_specs=[_row_spec(F_H), _row_spec(DW), _agg_spec(F_H),
                  _agg_spec(F_H), _full_spec((3 * F_H, F_H)),
                  _full_spec((1, F_H)), _full_spec((F_H, F_OUT)),
                  _full_spec((1, F_OUT))],
        out_specs=_row_spec(F_OUT),
        out_shape=jax.ShapeDtypeStruct((N_N, F_OUT), jnp.float32),
    )(h, dis, agg1, agg2, W3, b3r, W4, b4r)

    return y


# speedup vs baseline: 1.1282x; 1.0242x over previous
import functools

import jax
import jax.numpy as jnp
from jax import lax
from jax.experimental import pallas as pl
from jax.experimental.pallas import tpu as pltpu
import jax.experimental.pallas.tpu_sc as plsc

N_N = 10000
N_E = 320000
F_IN = 128
F_H = 64
F_OUT = 2

NC = 2
NS = 16
CH = 80
N_P = 10240
E_ROWS = N_E // CH
NCH = E_ROWS // (NC * NS)
RPW = N_P // NS
DW = 8
GC = 1
NG = NCH // GC

_SC_MESH = plsc.VectorSubcoreMesh(core_axis_name="c", subcore_axis_name="s")
_SC_PARAMS = pltpu.CompilerParams(use_tc_tiling_on_sc=False)


@functools.partial(
    pl.kernel,
    out_type=jax.ShapeDtypeStruct((NC, N_P, DW), jnp.float32),
    mesh=_SC_MESH,
    scratch_types=[
        pltpu.VMEM((NCH, CH), jnp.int32),
        pltpu.VMEM((CH, DW), jnp.float32),
        pltpu.VMEM_SHARED((N_P, DW), jnp.float32),
    ],
    compiler_params=_SC_PARAMS,
)
def _deg_sc(edge_hbm, zeros_hbm, ones_hbm, out_hbm, dst_v, ones_v, deg_sh):
    cid = lax.axis_index("c")
    sid = lax.axis_index("s")
    row0 = sid * RPW
    cbase = (cid * NS + sid) * NCH
    pltpu.sync_copy(zeros_hbm.at[pl.ds(row0, RPW)], deg_sh.at[pl.ds(row0, RPW)])
    pltpu.sync_copy(ones_hbm, ones_v)
    pltpu.sync_copy(edge_hbm.at[1, pl.ds(cbase, NCH)], dst_v)
    plsc.subcore_barrier()

    def body(i, carry):
        pltpu.sync_copy(ones_v, deg_sh.at[dst_v.at[i]], add=True)
        return carry

    lax.fori_loop(0, NCH, body, 0)
    plsc.subcore_barrier()
    pltpu.sync_copy(deg_sh.at[pl.ds(row0, RPW)], out_hbm.at[cid, pl.ds(row0, RPW)])


@functools.partial(
    pl.kernel,
    out_type=jax.ShapeDtypeStruct((NC, N_P, F_H), jnp.float32),
    mesh=_SC_MESH,
    scratch_types=[
        pltpu.VMEM((NCH, CH), jnp.int32),
        pltpu.VMEM((NCH, CH), jnp.int32),
        pltpu.VMEM((GC, CH, F_H), jnp.float32),
        pltpu.VMEM((GC, CH, F_H), jnp.float32),
        pltpu.VMEM_SHARED((N_P, F_H), jnp.float32),
        pltpu.VMEM_SHARED((N_P, F_H), jnp.float32),
        pltpu.SemaphoreType.DMA,
        pltpu.SemaphoreType.DMA,
        pltpu.SemaphoreType.DMA,
        pltpu.SemaphoreType.DMA,
    ],
    compiler_params=_SC_PARAMS,
)
def _prop_sc(g_hbm_in, edge_hbm, zeros_hbm, out_hbm,
             src_v, dst_v, rows0, rows1, g_hbm, agg_sh, sg0, sg1, ss0, ss1):
    cid = lax.axis_index("c")
    sid = lax.axis_index("s")
    row0 = sid * RPW
    cbase = (cid * NS + sid) * NCH
    pltpu.sync_copy(zeros_hbm.at[pl.ds(row0, RPW)], agg_sh.at[pl.ds(row0, RPW)])
    pltpu.sync_copy(g_hbm_in.at[pl.ds(row0, RPW)], g_hbm.at[pl.ds(row0, RPW)])
    pltpu.sync_copy(edge_hbm.at[0, pl.ds(cbase, NCH)], src_v)
    pltpu.sync_copy(edge_hbm.at[1, pl.ds(cbase, NCH)], dst_v)
    plsc.subcore_barrier()

    def gather_issue(g, rows, sg):
        def c_body(c, carry):
            pltpu.async_copy(g_hbm.at[src_v.at[g * GC + c]], rows.at[c], sg)
            return carry
        lax.fori_loop(0, GC, c_body, 0)

    def gather_wait(g, rows, sg):
        def c_body(c, carry):
            pltpu.make_async_copy(
                g_hbm.at[src_v.at[g * GC + c]], rows.at[c], sg).wait()
            return carry
        lax.fori_loop(0, GC, c_body, 0)

    def scatter_issue(g, rows, ss):
        def c_body(c, carry):
            pltpu.async_copy(
                rows.at[c], agg_sh.at[dst_v.at[g * GC + c]], ss, add=True)
            return carry
        lax.fori_loop(0, GC, c_body, 0)

    def scatter_wait(g, rows, ss):
        def c_body(c, carry):
            pltpu.make_async_copy(
                rows.at[c], agg_sh.at[dst_v.at[g * GC + c]], ss).wait()
            return carry
        lax.fori_loop(0, GC, c_body, 0)

    halves = ((rows0, sg0, ss0), (rows1, sg1, ss1))
    gather_issue(0, rows0, sg0)

    def pair(p, carry):
        for half in range(2):
            rows, sg, ss = halves[half]
            orows, osg, oss = halves[1 - half]
            g = 2 * p + half
            gather_wait(g, rows, sg)
            scatter_issue(g, rows, ss)

            @pl.when(g >= 1)
            def _():
                scatter_wait(g - 1, orows, oss)

            @pl.when(g + 1 < NG)
            def _():
                gather_issue(g + 1, orows, osg)
        return carry

    lax.fori_loop(0, NG // 2, pair, 0)
    gather_wait(NG - 1, rows0, sg0)
    scatter_issue(NG - 1, rows0, ss0)
    scatter_wait(NG - 2, rows1, ss1)
    scatter_wait(NG - 1, rows0, ss0)
    plsc.subcore_barrier()
    pltpu.sync_copy(agg_sh.at[pl.ds(row0, RPW)], out_hbm.at[cid, pl.ds(row0, RPW)])


BR = 2000


def _mlp_body(x_ref, w1_ref, b1_ref, w2_ref, b2_ref, o_ref):
    h1 = jnp.dot(x_ref[...], w1_ref[...], preferred_element_type=jnp.float32)
    h1 = jnp.maximum(h1 + b1_ref[...], 0.0)
    h2 = jnp.dot(h1, w2_ref[...], preferred_element_type=jnp.float32)
    o_ref[...] = jnp.maximum(h2 + b2_ref[...], 0.0)


def _scale_body(deg_ref, h_ref, g_ref, dis_ref):
    deg = jnp.maximum(deg_ref[0] + deg_ref[1], 1.0)
    dis = lax.rsqrt(deg)
    dis_ref[...] = dis
    g_ref[...] = h_ref[...] * dis[:, 0:1]


def _update_body(h_ref, dis_ref, agg_ref, g_ref):
    dis = dis_ref[:, 0:1]
    p = h_ref[...] - dis * (agg_ref[0] + agg_ref[1])
    g_ref[...] = p * dis


def _final_body(h_ref, dis_ref, agg1_ref, agg2_ref, w3_ref, b3_ref, w4_ref,
                b4_ref, y_ref):
    dis = dis_ref[:, 0:1]
    h = h_ref[...]
    q1 = dis * (agg1_ref[0] + agg1_ref[1])
    q2 = dis * (agg2_ref[0] + agg2_ref[1])
    c0 = 0.75 * h + 2.25 * q1 - 0.75 * q2
    c1 = 1.5 * h - 1.5 * q1 + 1.5 * q2
    c2 = 0.75 * h - 0.75 * q1 - 0.75 * q2
    acts = jnp.dot(c0, w3_ref[0:F_H, :], preferred_element_type=jnp.float32)
    acts += jnp.dot(c1, w3_ref[F_H:2 * F_H, :], preferred_element_type=jnp.float32)
    acts += jnp.dot(c2, w3_ref[2 * F_H:3 * F_H, :], preferred_element_type=jnp.float32)
    acts = jnp.maximum(acts + b3_ref[...], 0.0)
    y = jnp.dot(acts, w4_ref[...], preferred_element_type=jnp.float32)
    y_ref[...] = y + b4_ref[...]


def _row_spec(width):
    return pl.BlockSpec((BR, width), lambda i: (i, 0))


def _full_spec(shape):
    nd = len(shape)
    return pl.BlockSpec(shape, lambda i, _nd=nd: (0,) * _nd)


def _agg_spec(width):
    return pl.BlockSpec((NC, BR, width), lambda i: (0, i, 0))


def kernel(x, edge_index, W1, b1, W2, b2, W3, b3, W4, b4):
    edge3d = edge_index.astype(jnp.int32).reshape(2, E_ROWS, CH)
    zeros_h = jnp.zeros((N_P, F_H), jnp.float32)
    zeros_d = jnp.zeros((N_P, DW), jnp.float32)
    ones_d = jnp.ones((CH, DW), jnp.float32)
    b1r = b1.reshape(1, F_H)
    b2r = b2.reshape(1, F_H)
    b3r = b3.reshape(1, F_H)
    b4r = b4.reshape(1, F_OUT)

    grid = (N_N // BR,)

    deg2 = _deg_sc(edge3d, zeros_d, ones_d)

    h = pl.pallas_call(
        _mlp_body,
        grid=grid,
        in_specs=[_row_spec(F_IN), _full_spec((F_IN, F_H)), _full_spec((1, F_H)),
                  _full_spec((F_H, F_H)), _full_spec((1, F_H))],
        out_specs=_row_spec(F_H),
        out_shape=jax.ShapeDtypeStruct((N_N, F_H), jnp.float32),
    )(x, W1, b1r, W2, b2r)

    g1, dis = pl.pallas_call(
        _scale_body,
        grid=grid,
        in_specs=[_agg_spec(DW), _row_spec(F_H)],
        out_specs=[_row_spec(F_H), _row_spec(DW)],
        out_shape=[jax.ShapeDtypeStruct((N_P, F_H), jnp.float32),
                   jax.ShapeDtypeStruct((N_N, DW), jnp.float32)],
    )(deg2, h)

    agg1 = _prop_sc(g1, edge3d, zeros_h)

    g2 = pl.pallas_call(
        _update_body,
        grid=grid,
        in_specs=[_row_spec(F_H), _row_spec(DW), _agg_spec(F_H)],
        out_specs=_row_spec(F_H),
        out_shape=jax.ShapeDtypeStruct((N_P, F_H), jnp.float32),
    )(h, dis, agg1)

    agg2 = _prop_sc(g2, edge3d, zeros_h)

    y = pl.pallas_call(
        _final_body,
        grid=grid,
        in_specs=[_row_spec(F_H), _row_spec(DW), _agg_spec(F_H),
                  _agg_spec(F_H), _full_spec((3 * F_H, F_H)),
                  _full_spec((1, F_H)), _full_spec((F_H, F_OUT)),
                  _full_spec((1, F_OUT))],
        out_specs=_row_spec(F_OUT),
        out_shape=jax.ShapeDtypeStruct((N_N, F_OUT), jnp.float32),
    )(h, dis, agg1, agg2, W3, b3r, W4, b4r)

    return y
